# baseline (speedup 1.0000x reference)
import functools

import jax
import jax.numpy as jnp
from jax import lax
from jax.experimental import pallas as pl
from jax.experimental.pallas import tpu as pltpu
from jax.experimental.pallas import tpu_sc as plsc

_HIDDEN = 768
_MAX_POS = 512
_TYPE_VOCAB = 2
_NTOK = 16 * 512
_NW = 32
_TPW = _NTOK // _NW
_CH = 16
_NCH = _TPW // _CH


def _rsqrt_newton(x):
    i = lax.bitcast_convert_type(x, jnp.int32)
    i = jnp.int32(0x5F3759DF) - lax.shift_right_logical(i, 1)
    y = lax.bitcast_convert_type(i, jnp.float32)
    for _ in range(3):
        y = y * (1.5 - 0.5 * x * y * y)
    return y


_mesh = plsc.VectorSubcoreMesh(core_axis_name="c", subcore_axis_name="s")


def _body(ids_hbm, tt_hbm, word_hbm, pos_hbm, typ_hbm, g_hbm, b_hbm,
          out_hbm, idx_v, tt_v, rows_v, pos_v, typ_v, sum_v, g_v, b_v,
          sem):
    wid = lax.axis_index("s") * 2 + lax.axis_index("c")
    base = wid * _TPW
    pltpu.sync_copy(g_hbm, g_v)
    pltpu.sync_copy(b_hbm, b_v)
    pltpu.sync_copy(typ_hbm, typ_v)
    lanes = lax.iota(jnp.int32, 16)
    zero = jnp.zeros((16,), jnp.float32)

    def chunk_body(c, carry):
        tok0 = base + c * _CH
        pltpu.sync_copy(ids_hbm.at[pl.ds(tok0, _CH)], idx_v)
        pltpu.sync_copy(tt_hbm.at[pl.ds(tok0, _CH)], tt_v)
        pltpu.async_copy(word_hbm.at[idx_v], rows_v, sem).wait()
        pltpu.sync_copy(pos_hbm.at[pl.ds(lax.rem(tok0, _MAX_POS), _CH)],
                        pos_v)
        ttvec = tt_v[...]

        def h_body(h, hcarry):
            acc, acc2 = hcarry
            hb = jnp.zeros((16,), jnp.int32) + h
            wv = plsc.load_gather(rows_v, [lanes, hb])
            pv = plsc.load_gather(pos_v, [lanes, hb])
            tv = plsc.load_gather(typ_v, [ttvec, hb])
            sv = wv + pv + tv
            sum_v[pl.ds(h * 16, 16)] = sv
            return acc + sv, acc2 + sv * sv

        acc, acc2 = lax.fori_loop(0, _HIDDEN, h_body, (zero, zero))
        mean = acc * (1.0 / _HIDDEN)
        var = acc2 * (1.0 / _HIDDEN) - mean * mean
        inv = _rsqrt_newton(var + 1e-12)

        def h2_body(h, h2carry):
            hb = jnp.zeros((16,), jnp.int32) + h
            sv = sum_v[pl.ds(h * 16, 16)]
            gb = plsc.load_gather(g_v, [hb])
            bb = plsc.load_gather(b_v, [hb])
            o = (sv - mean) * inv * gb + bb
            plsc.store_scatter(rows_v, [lanes, hb], o)
            return h2carry

        lax.fori_loop(0, _HIDDEN, h2_body, 0)
        pltpu.sync_copy(rows_v, out_hbm.at[pl.ds(tok0, _CH)])
        return carry

    lax.fori_loop(0, _NCH, chunk_body, 0)


def _build(interpret=False):
    return functools.partial(
        pl.kernel,
        mesh=_mesh,
        compiler_params=pltpu.CompilerParams(needs_layout_passes=False),
        out_type=jax.ShapeDtypeStruct((_NTOK, _HIDDEN), jnp.float32),
        interpret=interpret,
        scratch_types=[
            pltpu.VMEM((_CH,), jnp.int32),
            pltpu.VMEM((_CH,), jnp.int32),
            pltpu.VMEM((_CH, _HIDDEN), jnp.float32),
            pltpu.VMEM((_CH, _HIDDEN), jnp.float32),
            pltpu.VMEM((_TYPE_VOCAB, _HIDDEN), jnp.float32),
            pltpu.VMEM((_CH * _HIDDEN,), jnp.float32),
            pltpu.VMEM((_HIDDEN,), jnp.float32),
            pltpu.VMEM((_HIDDEN,), jnp.float32),
            pltpu.SemaphoreType.DMA,
        ],
    )(_body)


_bert_emb = _build()


def kernel(input_ids, token_type_ids, word_embeddings, position_embeddings,
           token_type_embeddings, ln_gamma, ln_beta):
    ids = input_ids.reshape(-1).astype(jnp.int32)
    tt = token_type_ids.reshape(-1).astype(jnp.int32)
    out = _bert_emb(ids, tt, word_embeddings, position_embeddings,
                    token_type_embeddings, ln_gamma, ln_beta)
    return out.reshape(input_ids.shape[0], input_ids.shape[1], _HIDDEN)
